# Initial kernel scaffold; baseline (speedup 1.0000x reference)
import jax
import jax.numpy as jnp
from jax.experimental import pallas as pl


def kernel(bonds, bond_atom_1, bond_atom_2, atoms, state, W_e1, b_e1, W_e2, b_e2, W_e3, b_e3, W_v1, b_v1, W_v2, b_v2, W_v3, b_v3, W_u1, b_u1, W_u2, b_u2, W_u3, b_u3):
    raise NotImplementedError("write your pallas kernel here")


# speedup vs baseline: 247.5075x; 247.5075x over previous
import functools

import jax
import jax.numpy as jnp
from jax import lax
from jax.experimental import pallas as pl
from jax.experimental.pallas import tpu as pltpu
from jax.experimental.pallas import tpu_sc as plsc

_NC = 2
_NS = 16
_NW = _NC * _NS


def _softplus(x):
    return jnp.maximum(x, 0.0) + jnp.log1p(jnp.exp(-jnp.abs(x)))


def _gather_body(n, nblk, blk, sub, atoms_hbm, idx1_hbm, idx2_hbm,
                 out1_hbm, out2_hbm, idx_v, rows_v, atoms_sp, sem):
    c = lax.axis_index("c")
    s = lax.axis_index("s")
    wid = s * _NC + c
    ew = nblk * blk
    base = wid * ew

    chunk = (n // _NS) // 8 * 8
    rem = n - chunk * _NS
    pltpu.sync_copy(atoms_hbm.at[pl.ds(s * chunk, chunk)],
                    rows_v.at[pl.ds(0, chunk)])
    pltpu.sync_copy(rows_v.at[pl.ds(0, chunk)],
                    atoms_sp.at[pl.ds(s * chunk, chunk)])
    if rem:
        @pl.when(s == 0)
        def _():
            pltpu.sync_copy(atoms_hbm.at[pl.ds(chunk * _NS, rem)],
                            rows_v.at[pl.ds(0, rem)])
            pltpu.sync_copy(rows_v.at[pl.ds(0, rem)],
                            atoms_sp.at[pl.ds(chunk * _NS, rem)])
    plsc.subcore_barrier()

    def one_side(idx_hbm, out_hbm):
        def block(t, carry):
            off = base + t * blk
            pltpu.sync_copy(idx_hbm.at[pl.ds(off, blk)], idx_v)
            descs = [
                pltpu.async_copy(
                    atoms_sp.at[idx_v.at[pl.ds(j * sub, sub)]],
                    rows_v.at[pl.ds(j * sub, sub)],
                    sem,
                )
                for j in range(blk // sub)
            ]
            for d in descs:
                d.wait()
            pltpu.sync_copy(rows_v, out_hbm.at[pl.ds(off, blk)])
            return carry

        lax.fori_loop(0, nblk, block, 0)

    one_side(idx1_hbm, out1_hbm)
    one_side(idx2_hbm, out2_hbm)


def _sc_gather(atoms2, idx1, idx2):
    n, d = atoms2.shape
    e = idx1.shape[0]
    ew = e // _NW
    assert ew * _NW == e
    blk = 2000
    sub = 80
    nblk = ew // blk
    assert nblk * blk == ew
    mesh = plsc.VectorSubcoreMesh(core_axis_name="c", subcore_axis_name="s")
    f = pl.kernel(
        functools.partial(_gather_body, n, nblk, blk, sub),
        out_type=(
            jax.ShapeDtypeStruct((e, d), jnp.float32),
            jax.ShapeDtypeStruct((e, d), jnp.float32),
        ),
        mesh=mesh,
        scratch_types=[
            pltpu.VMEM((blk,), jnp.int32),
            pltpu.VMEM((blk, d), jnp.float32),
            pltpu.VMEM_SHARED((n, d), jnp.float32),
            pltpu.SemaphoreType.DMA,
        ],
        compiler_params=pltpu.CompilerParams(use_tc_tiling_on_sc=False),
    )
    return f(atoms2, idx1, idx2)


def _scatter_body(n, rows_pc, k, nit, extra, zeros32_hbm, zeros16_hbm,
                  ones_hbm, idx_hbm, vals_hbm, sums_out, counts_out,
                  idx_v, vals_v, ones_v, z16_v, sums_sp, counts_sp, sem):
    c = lax.axis_index("c")
    s = lax.axis_index("s")
    chunk = (n // _NS) // 8 * 8
    rem = n - chunk * _NS

    pltpu.sync_copy(ones_hbm, ones_v)

    def zero_slice(off, ln):
        pltpu.sync_copy(zeros32_hbm.at[pl.ds(off, ln)],
                        vals_v.at[pl.ds(0, ln)])
        pltpu.sync_copy(vals_v.at[pl.ds(0, ln)], sums_sp.at[pl.ds(off, ln)])
        pltpu.sync_copy(zeros16_hbm.at[pl.ds(off, ln)],
                        z16_v.at[pl.ds(0, ln)])
        pltpu.sync_copy(z16_v.at[pl.ds(0, ln)],
                        counts_sp.at[pl.ds(off, ln)])

    zero_slice(s * chunk, chunk)
    if rem:
        @pl.when(s == 0)
        def _():
            zero_slice(chunk * _NS, rem)
    plsc.subcore_barrier()

    base_row = c * rows_pc + s * (nit * k)

    def block(t, carry):
        r0 = base_row + t * k
        pltpu.sync_copy(idx_hbm.at[pl.ds(r0 * 128, k * 128)], idx_v)
        pltpu.sync_copy(vals_hbm.at[pl.ds(r0 * 128, k * 128)], vals_v)
        descs = []
        for j in range(k):
            descs.append(pltpu.async_copy(
                vals_v.at[pl.ds(j * 128, 128)],
                sums_sp.at[idx_v.at[pl.ds(j * 128, 128)]], sem, add=True))
            descs.append(pltpu.async_copy(
                ones_v, counts_sp.at[idx_v.at[pl.ds(j * 128, 128)]], sem,
                add=True))
        for dsc in descs:
            dsc.wait()
        return carry

    lax.fori_loop(0, nit, block, 0)

    @pl.when(s < extra)
    def _():
        r0 = c * rows_pc + nit * k * _NS + s
        pltpu.sync_copy(idx_hbm.at[pl.ds(r0 * 128, 128)],
                        idx_v.at[pl.ds(0, 128)])
        pltpu.sync_copy(vals_hbm.at[pl.ds(r0 * 128, 128)],
                        vals_v.at[pl.ds(0, 128)])
        d1 = pltpu.async_copy(vals_v.at[pl.ds(0, 128)],
                              sums_sp.at[idx_v.at[pl.ds(0, 128)]], sem,
                              add=True)
        d2 = pltpu.async_copy(ones_v, counts_sp.at[idx_v.at[pl.ds(0, 128)]],
                              sem, add=True)
        d1.wait()
        d2.wait()

    plsc.subcore_barrier()

    def out_slice(off, ln):
        pltpu.sync_copy(sums_sp.at[pl.ds(off, ln)], vals_v.at[pl.ds(0, ln)])
        pltpu.sync_copy(vals_v.at[pl.ds(0, ln)],
                        sums_out.at[c].at[pl.ds(off, ln)])
        pltpu.sync_copy(counts_sp.at[pl.ds(off, ln)], z16_v.at[pl.ds(0, ln)])
        pltpu.sync_copy(z16_v.at[pl.ds(0, ln)],
                        counts_out.at[c].at[pl.ds(off, ln)])

    out_slice(s * chunk, chunk)
    if rem:
        @pl.when(s == 0)
        def _():
            out_slice(chunk * _NS, rem)


def _sc_scatter(n, idx1, vals):
    e = vals.shape[0]
    d = vals.shape[1]
    nrows = e // 128
    assert nrows * 128 == e
    rows_pc = nrows // _NC
    assert rows_pc * _NC == nrows
    rows_pt = rows_pc // _NS
    extra = rows_pc - rows_pt * _NS
    k = 13
    nit = rows_pt // k
    assert nit * k == rows_pt, (rows_pt, k)
    chunk = (n // _NS) // 8 * 8
    rem = n - chunk * _NS
    stage = max(k * 128, chunk + rem)
    mesh = plsc.VectorSubcoreMesh(core_axis_name="c", subcore_axis_name="s")
    f = pl.kernel(
        functools.partial(_scatter_body, n, rows_pc, k, nit, extra),
        out_type=(
            jax.ShapeDtypeStruct((_NC, n, d), jnp.float32),
            jax.ShapeDtypeStruct((_NC, n, 16), jnp.float32),
        ),
        mesh=mesh,
        scratch_types=[
            pltpu.VMEM((k * 128,), jnp.int32),
            pltpu.VMEM((stage, d), jnp.float32),
            pltpu.VMEM((128, 16), jnp.float32),
            pltpu.VMEM((stage, 16), jnp.float32),
            pltpu.VMEM_SHARED((n, d), jnp.float32),
            pltpu.VMEM_SHARED((n, 16), jnp.float32),
            pltpu.SemaphoreType.DMA,
        ],
        compiler_params=pltpu.CompilerParams(use_tc_tiling_on_sc=False),
    )
    zeros32 = jnp.zeros((n, d), jnp.float32)
    zeros16 = jnp.zeros((n, 16), jnp.float32)
    ones = jnp.ones((128, 16), jnp.float32)
    return f(zeros32, zeros16, ones, idx1, vals)


def _edge_body(a1_ref, a2_ref, b_ref, st_ref,
               w1a_ref, w1b_ref, w1c_ref, w1d_ref, b1_ref,
               w2_ref, b2_ref, w3_ref, b3_ref,
               out_ref, bsum_ref):
    i = pl.program_id(0)
    st_term = jnp.dot(st_ref[...], w1d_ref[...],
                      preferred_element_type=jnp.float32) + b1_ref[...]
    x = (jnp.dot(a1_ref[...], w1a_ref[...], preferred_element_type=jnp.float32)
         + jnp.dot(a2_ref[...], w1b_ref[...], preferred_element_type=jnp.float32)
         + jnp.dot(b_ref[...], w1c_ref[...], preferred_element_type=jnp.float32)
         + st_term)
    h = _softplus(x)
    h = _softplus(jnp.dot(h, w2_ref[...], preferred_element_type=jnp.float32)
                  + b2_ref[...])
    o = _softplus(jnp.dot(h, w3_ref[...], preferred_element_type=jnp.float32)
                  + b3_ref[...])
    out_ref[...] = o

    @pl.when(i == 0)
    def _():
        bsum_ref[...] = jnp.zeros_like(bsum_ref)

    bsum_ref[0:1, :] += jnp.sum(o, axis=0, keepdims=True)


def _tc_edge_mlp(a1g, a2g, bonds2, st_row, w1a, w1b, w1c, w1d, b1, w2, b2,
                 w3, b3):
    e, d = bonds2.shape
    r = 1280
    grid = e // r
    assert grid * r == e
    row_spec = pl.BlockSpec((r, d), lambda i: (i, 0))
    full = pl.BlockSpec(lambda i: (0, 0))

    def fs(x):
        return pl.BlockSpec(x.shape, lambda i: tuple(0 for _ in x.shape))

    out, bsum = pl.pallas_call(
        _edge_body,
        grid=(grid,),
        in_specs=[row_spec, row_spec, row_spec, fs(st_row),
                  fs(w1a), fs(w1b), fs(w1c), fs(w1d), fs(b1),
                  fs(w2), fs(b2), fs(w3), fs(b3)],
        out_specs=[pl.BlockSpec((r, 32), lambda i: (i, 0)),
                   pl.BlockSpec((8, 32), lambda i: (0, 0))],
        out_shape=[jax.ShapeDtypeStruct((e, 32), jnp.float32),
                   jax.ShapeDtypeStruct((8, 32), jnp.float32)],
    )(a1g, a2g, bonds2, st_row, w1a, w1b, w1c, w1d, b1, w2, b2, w3, b3)
    return out, bsum


def _node_body(e_edges, sums_ref, counts_ref, atoms_ref, st_ref, bsum_ref,
               wv1a_ref, wv1b_ref, wv1c_ref, bv1_ref, wv2_ref, bv2_ref,
               wv3_ref, bv3_ref,
               wu1a_ref, wu1b_ref, wu1c_ref, bu1_ref, wu2_ref, bu2_ref,
               wu3_ref, bu3_ref,
               atoms_out_ref, state_out_ref):
    n = atoms_ref.shape[0]
    ssum = sums_ref[0] + sums_ref[1]
    cnt = counts_ref[0, :, 0:1] + counts_ref[1, :, 0:1]
    bta = ssum / cnt
    st = st_ref[...]
    x = (jnp.dot(bta, wv1a_ref[...], preferred_element_type=jnp.float32)
         + jnp.dot(atoms_ref[...], wv1b_ref[...],
                   preferred_element_type=jnp.float32)
         + jnp.dot(st, wv1c_ref[...], preferred_element_type=jnp.float32)
         + bv1_ref[...])
    h = _softplus(x)
    h = _softplus(jnp.dot(h, wv2_ref[...], preferred_element_type=jnp.float32)
                  + bv2_ref[...])
    atoms_new = _softplus(
        jnp.dot(h, wv3_ref[...], preferred_element_type=jnp.float32)
        + bv3_ref[...])
    atoms_out_ref[...] = atoms_new

    bts = bsum_ref[0:1, :] * (1.0 / e_edges)
    ats = jnp.sum(atoms_new, axis=0, keepdims=True) * (1.0 / n)
    xs = (jnp.dot(bts, wu1a_ref[...], preferred_element_type=jnp.float32)
          + jnp.dot(ats, wu1b_ref[...], preferred_element_type=jnp.float32)
          + jnp.dot(st, wu1c_ref[...], preferred_element_type=jnp.float32)
          + bu1_ref[...])
    hs = _softplus(xs)
    hs = _softplus(jnp.dot(hs, wu2_ref[...],
                           preferred_element_type=jnp.float32) + bu2_ref[...])
    sn = _softplus(jnp.dot(hs, wu3_ref[...],
                           preferred_element_type=jnp.float32) + bu3_ref[...])
    state_out_ref[...] = jnp.broadcast_to(sn, state_out_ref.shape)


def _tc_node_state(e_edges, sums, counts, atoms2, st_row, bsum, params):
    n, d = atoms2.shape

    def fs(x):
        return pl.BlockSpec(x.shape, lambda: tuple(0 for _ in x.shape))

    args = (sums, counts, atoms2, st_row, bsum) + tuple(params)
    atoms_new, state_new = pl.pallas_call(
        functools.partial(_node_body, e_edges),
        grid=(),
        in_specs=[fs(a) for a in args],
        out_specs=[pl.BlockSpec((n, 32), lambda: (0, 0)),
                   pl.BlockSpec((8, 32), lambda: (0, 0))],
        out_shape=[jax.ShapeDtypeStruct((n, 32), jnp.float32),
                   jax.ShapeDtypeStruct((8, 32), jnp.float32)],
    )(*args)
    return atoms_new, state_new


def kernel(bonds, bond_atom_1, bond_atom_2, atoms, state,
           W_e1, b_e1, W_e2, b_e2, W_e3, b_e3,
           W_v1, b_v1, W_v2, b_v2, W_v3, b_v3,
           W_u1, b_u1, W_u2, b_u2, W_u3, b_u3):
    b, e, d = bonds.shape
    n = atoms.shape[1]
    bonds2 = bonds.reshape(e, d)
    atoms2 = atoms.reshape(n, d)
    idx1 = bond_atom_1.reshape(e).astype(jnp.int32)
    idx2 = bond_atom_2.reshape(e).astype(jnp.int32)
    st_row = state.reshape(1, d)

    a1g, a2g = _sc_gather(atoms2, idx1, idx2)

    b1r = b_e1.reshape(1, -1)
    bonds_new, bsum = _tc_edge_mlp(
        a1g, a2g, bonds2, st_row,
        W_e1[0:32], W_e1[32:64], W_e1[64:96], W_e1[96:128], b1r,
        W_e2, b_e2.reshape(1, -1), W_e3, b_e3.reshape(1, -1))

    sums, counts = _sc_scatter(n, idx1, bonds_new)

    node_params = (
        W_v1[0:32], W_v1[32:64], W_v1[64:96], b_v1.reshape(1, -1),
        W_v2, b_v2.reshape(1, -1), W_v3, b_v3.reshape(1, -1),
        W_u1[0:32], W_u1[32:64], W_u1[64:96], b_u1.reshape(1, -1),
        W_u2, b_u2.reshape(1, -1), W_u3, b_u3.reshape(1, -1))
    atoms_new, state_new = _tc_node_state(
        float(e), sums, counts, atoms2, st_row, bsum, node_params)

    return (bonds_new.reshape(b, e, 32),
            atoms_new.reshape(b, n, 32),
            state_new[0:1, :].reshape(b, 1, 32))


# speedup vs baseline: 459.7690x; 1.8576x over previous
import functools

import jax
import jax.numpy as jnp
from jax import lax
from jax.experimental import pallas as pl
from jax.experimental.pallas import tpu as pltpu
from jax.experimental.pallas import tpu_sc as plsc

_NC = 2
_NS = 16
_NW = _NC * _NS


def _softplus(x):
    return jnp.maximum(x, 0.0) + jnp.log1p(jnp.exp(-jnp.abs(x)))


def _bd4(w):
    a, b = w.shape
    z = jnp.zeros((a, b), w.dtype)
    rows = [jnp.concatenate([w if i == j else z for j in range(4)], axis=1)
            for i in range(4)]
    return jnp.concatenate(rows, axis=0)


def _gather_body(n, nblk, blk, sub, atoms_hbm, idx1_hbm, idx2_hbm,
                 out1_hbm, out2_hbm, idx_v, rows_v, atoms_sp, sem):
    c = lax.axis_index("c")
    s = lax.axis_index("s")
    wid = s * _NC + c
    ew = nblk * blk
    base = wid * ew

    chunk = (n // _NS) // 8 * 8
    rem = n - chunk * _NS
    pltpu.sync_copy(atoms_hbm.at[pl.ds(s * chunk, chunk)],
                    rows_v.at[pl.ds(0, chunk)])
    pltpu.sync_copy(rows_v.at[pl.ds(0, chunk)],
                    atoms_sp.at[pl.ds(s * chunk, chunk)])
    if rem:
        @pl.when(s == 0)
        def _():
            pltpu.sync_copy(atoms_hbm.at[pl.ds(chunk * _NS, rem)],
                            rows_v.at[pl.ds(0, rem)])
            pltpu.sync_copy(rows_v.at[pl.ds(0, rem)],
                            atoms_sp.at[pl.ds(chunk * _NS, rem)])
    plsc.subcore_barrier()

    def one_side(idx_hbm, out_hbm):
        def block(t, carry):
            off = base + t * blk
            pltpu.sync_copy(idx_hbm.at[pl.ds(off, blk)], idx_v)
            descs = [
                pltpu.async_copy(
                    atoms_sp.at[idx_v.at[pl.ds(j * sub, sub)]],
                    rows_v.at[pl.ds(j * sub, sub)],
                    sem,
                )
                for j in range(blk // sub)
            ]
            for d in descs:
                d.wait()
            pltpu.sync_copy(rows_v, out_hbm.at[pl.ds(off, blk)])
            return carry

        lax.fori_loop(0, nblk, block, 0)

    one_side(idx1_hbm, out1_hbm)
    one_side(idx2_hbm, out2_hbm)


def _sc_gather(atoms2, idx1, idx2):
    n, d = atoms2.shape
    e = idx1.shape[0]
    ew = e // _NW
    assert ew * _NW == e
    blk = 2000
    sub = 80
    nblk = ew // blk
    assert nblk * blk == ew
    mesh = plsc.VectorSubcoreMesh(core_axis_name="c", subcore_axis_name="s")
    f = pl.kernel(
        functools.partial(_gather_body, n, nblk, blk, sub),
        out_type=(
            jax.ShapeDtypeStruct((e, d), jnp.float32),
            jax.ShapeDtypeStruct((e, d), jnp.float32),
        ),
        mesh=mesh,
        scratch_types=[
            pltpu.VMEM((blk,), jnp.int32),
            pltpu.VMEM((blk, d), jnp.float32),
            pltpu.VMEM_SHARED((n, d), jnp.float32),
            pltpu.SemaphoreType.DMA,
        ],
        compiler_params=pltpu.CompilerParams(use_tc_tiling_on_sc=False),
    )
    return f(atoms2, idx1, idx2)


def _scatter_body(n, rows_pc, k, nit, extra, zeros32_hbm,
                  ones_hbm, idx_hbm, vals_hbm, sums_out, counts_out,
                  idx_v, vals_v, ones_v, sums_sp, counts_sp, sem):
    c = lax.axis_index("c")
    s = lax.axis_index("s")
    chunk = (n // _NS) // 8 * 8
    rem = n - chunk * _NS

    pltpu.sync_copy(ones_hbm, ones_v)

    def zero_slice(off, ln):
        pltpu.sync_copy(zeros32_hbm.at[pl.ds(off, ln)],
                        vals_v.at[pl.ds(0, ln)])
        pltpu.sync_copy(vals_v.at[pl.ds(0, ln)], sums_sp.at[pl.ds(off, ln)])
        pltpu.sync_copy(vals_v.at[pl.ds(0, ln)],
                        counts_sp.at[pl.ds(off, ln)])

    zero_slice(s * chunk, chunk)
    if rem:
        @pl.when(s == 0)
        def _():
            zero_slice(chunk * _NS, rem)
    plsc.subcore_barrier()

    base_row = c * rows_pc + s * (nit * k)

    def block(t, carry):
        r0 = base_row + t * k
        pltpu.sync_copy(idx_hbm.at[pl.ds(r0 * 128, k * 128)], idx_v)
        pltpu.sync_copy(vals_hbm.at[pl.ds(r0 * 128, k * 128)], vals_v)
        descs = []
        for j in range(k):
            descs.append(pltpu.async_copy(
                vals_v.at[pl.ds(j * 128, 128)],
                sums_sp.at[idx_v.at[pl.ds(j * 128, 128)]], sem, add=True))
            descs.append(pltpu.async_copy(
                ones_v, counts_sp.at[idx_v.at[pl.ds(j * 128, 128)]], sem,
                add=True))
        for dsc in descs:
            dsc.wait()
        return carry

    lax.fori_loop(0, nit, block, 0)

    @pl.when(s < extra)
    def _():
        r0 = c * rows_pc + nit * k * _NS + s
        pltpu.sync_copy(idx_hbm.at[pl.ds(r0 * 128, 128)],
                        idx_v.at[pl.ds(0, 128)])
        pltpu.sync_copy(vals_hbm.at[pl.ds(r0 * 128, 128)],
                        vals_v.at[pl.ds(0, 128)])
        d1 = pltpu.async_copy(vals_v.at[pl.ds(0, 128)],
                              sums_sp.at[idx_v.at[pl.ds(0, 128)]], sem,
                              add=True)
        d2 = pltpu.async_copy(ones_v, counts_sp.at[idx_v.at[pl.ds(0, 128)]],
                              sem, add=True)
        d1.wait()
        d2.wait()

    plsc.subcore_barrier()

    def out_slice(off, ln):
        pltpu.sync_copy(sums_sp.at[pl.ds(off, ln)], vals_v.at[pl.ds(0, ln)])
        pltpu.sync_copy(vals_v.at[pl.ds(0, ln)],
                        sums_out.at[c].at[pl.ds(off, ln)])
        pltpu.sync_copy(counts_sp.at[pl.ds(off, ln)],
                        vals_v.at[pl.ds(0, ln)])
        pltpu.sync_copy(vals_v.at[pl.ds(0, ln)],
                        counts_out.at[c].at[pl.ds(off, ln)])

    out_slice(s * chunk, chunk)
    if rem:
        @pl.when(s == 0)
        def _():
            out_slice(chunk * _NS, rem)


def _sc_scatter(n, idx1, vals):
    e = vals.shape[0]
    d = vals.shape[1]
    nrows = e // 128
    assert nrows * 128 == e
    rows_pc = nrows // _NC
    assert rows_pc * _NC == nrows
    rows_pt = rows_pc // _NS
    extra = rows_pc - rows_pt * _NS
    k = 13
    nit = rows_pt // k
    assert nit * k == rows_pt, (rows_pt, k)
    chunk = (n // _NS) // 8 * 8
    rem = n - chunk * _NS
    stage = max(k * 128, chunk + rem)
    mesh = plsc.VectorSubcoreMesh(core_axis_name="c", subcore_axis_name="s")
    f = pl.kernel(
        functools.partial(_scatter_body, n, rows_pc, k, nit, extra),
        out_type=(
            jax.ShapeDtypeStruct((_NC, n, d), jnp.float32),
            jax.ShapeDtypeStruct((_NC, n, d), jnp.float32),
        ),
        mesh=mesh,
        scratch_types=[
            pltpu.VMEM((k * 128,), jnp.int32),
            pltpu.VMEM((stage, d), jnp.float32),
            pltpu.VMEM((128, d), jnp.float32),
            pltpu.VMEM_SHARED((n, d), jnp.float32),
            pltpu.VMEM_SHARED((n, d), jnp.float32),
            pltpu.SemaphoreType.DMA,
        ],
        compiler_params=pltpu.CompilerParams(use_tc_tiling_on_sc=False),
    )
    zeros32 = jnp.zeros((n, d), jnp.float32)
    ones = jnp.ones((128, d), jnp.float32)
    return f(zeros32, ones, idx1, vals)


def _edge_body(g1_ref, g2_ref, b_ref, st_ref,
               w1a_ref, w1b_ref, w1c_ref, w1d_ref, b1_ref,
               w2_ref, b2_ref, w3_ref, b3_ref,
               out_ref, bsum_ref):
    i = pl.program_id(0)
    bf = jnp.bfloat16
    stt = jnp.dot(st_ref[...], w1d_ref[...],
                  preferred_element_type=jnp.float32) + b1_ref[...]
    stt4 = jnp.concatenate([stt, stt, stt, stt], axis=1)
    x = (jnp.dot(g1_ref[...].astype(bf), w1a_ref[...],
                 preferred_element_type=jnp.float32)
         + jnp.dot(g2_ref[...].astype(bf), w1b_ref[...],
                   preferred_element_type=jnp.float32)
         + jnp.dot(b_ref[...].astype(bf), w1c_ref[...],
                   preferred_element_type=jnp.float32)
         + stt4)
    h = _softplus(x)
    h = _softplus(jnp.dot(h.astype(bf), w2_ref[...],
                          preferred_element_type=jnp.float32) + b2_ref[...])
    o = _softplus(jnp.dot(h.astype(bf), w3_ref[...],
                          preferred_element_type=jnp.float32) + b3_ref[...])
    out_ref[...] = o

    @pl.when(i == 0)
    def _():
        bsum_ref[...] = jnp.zeros_like(bsum_ref)

    bsum_ref[0:1, :] += jnp.sum(o, axis=0, keepdims=True)


def _tc_edge_mlp(g1p, g2p, bondsp, st_row, w1a, w1b, w1c, w1d, b1, w2, b2,
                 w3, b3):
    e4 = bondsp.shape[0]
    bf = jnp.bfloat16
    w1a_bd = _bd4(w1a).astype(bf)
    w1b_bd = _bd4(w1b).astype(bf)
    w1c_bd = _bd4(w1c).astype(bf)
    b2t = jnp.tile(b2, 4)[None, :]
    w2_bd = _bd4(w2).astype(bf)
    w3_bd = _bd4(w3).astype(bf)
    b3t = jnp.tile(b3, 4)[None, :]
    r = 800
    grid = e4 // r
    assert grid * r == e4
    row_spec = pl.BlockSpec((r, 128), lambda i: (i, 0))

    def fs(x):
        return pl.BlockSpec(x.shape, lambda i: tuple(0 for _ in x.shape))

    args = (g1p, g2p, bondsp, st_row, w1a_bd, w1b_bd, w1c_bd, w1d,
            b1[None, :], w2_bd, b2t, w3_bd, b3t)
    out, bsum = pl.pallas_call(
        _edge_body,
        grid=(grid,),
        in_specs=[row_spec, row_spec, row_spec] + [fs(a) for a in args[3:]],
        out_specs=[pl.BlockSpec((r, 128), lambda i: (i, 0)),
                   pl.BlockSpec((8, 128), lambda i: (0, 0))],
        out_shape=[jax.ShapeDtypeStruct((e4, 128), jnp.float32),
                   jax.ShapeDtypeStruct((8, 128), jnp.float32)],
    )(*args)
    return out, bsum


def _node_body(e_edges, n_atoms, sums_ref, counts_ref, atoms_ref, st_ref,
               bsum_ref, fold_ref,
               wv1a_ref, wv1b_ref, wv1c_ref, bv1_ref, wv2_ref, bv2_ref,
               wv3_ref, bv3_ref,
               wu1a_ref, wu1b_ref, wu1c_ref, bu1_ref, wu2_ref, bu2_ref,
               wu3_ref, bu3_ref,
               atoms_out_ref, state_out_ref):
    bf = jnp.bfloat16
    ssum = sums_ref[0] + sums_ref[1]
    cnt = counts_ref[0] + counts_ref[1]
    bta = ssum / cnt
    st = st_ref[...]
    stt = (jnp.dot(st, wv1c_ref[...], preferred_element_type=jnp.float32)
           + bv1_ref[...])
    stt4 = jnp.concatenate([stt, stt, stt, stt], axis=1)
    x = (jnp.dot(bta.astype(bf), wv1a_ref[...],
                 preferred_element_type=jnp.float32)
         + jnp.dot(atoms_ref[...].astype(bf), wv1b_ref[...],
                   preferred_element_type=jnp.float32)
         + stt4)
    h = _softplus(x)
    h = _softplus(jnp.dot(h.astype(bf), wv2_ref[...],
                          preferred_element_type=jnp.float32) + bv2_ref[...])
    atoms_new = _softplus(
        jnp.dot(h.astype(bf), wv3_ref[...],
                preferred_element_type=jnp.float32) + bv3_ref[...])
    atoms_out_ref[...] = atoms_new

    fold = fold_ref[...]
    asum = jnp.dot(jnp.sum(atoms_new, axis=0, keepdims=True), fold,
                   preferred_element_type=jnp.float32)
    bsum = jnp.dot(jnp.sum(bsum_ref[...], axis=0, keepdims=True), fold,
                   preferred_element_type=jnp.float32)
    bts = bsum * (1.0 / e_edges)
    ats = asum * (1.0 / n_atoms)
    xs = (jnp.dot(bts, wu1a_ref[...], preferred_element_type=jnp.float32)
          + jnp.dot(ats, wu1b_ref[...], preferred_element_type=jnp.float32)
          + jnp.dot(st, wu1c_ref[...], preferred_element_type=jnp.float32)
          + bu1_ref[...])
    hs = _softplus(xs)
    hs = _softplus(jnp.dot(hs, wu2_ref[...],
                           preferred_element_type=jnp.float32) + bu2_ref[...])
    sn = _softplus(jnp.dot(hs, wu3_ref[...],
                           preferred_element_type=jnp.float32) + bu3_ref[...])
    state_out_ref[...] = jnp.broadcast_to(sn, state_out_ref.shape)


def _tc_node_state(e_edges, n_atoms, sums_p, counts_p, atoms_p, st_row,
                   bsum, params):
    n4 = atoms_p.shape[0]
    fold = jnp.tile(jnp.eye(32, dtype=jnp.float32), (4, 1))

    def fs(x):
        return pl.BlockSpec(x.shape, lambda: tuple(0 for _ in x.shape))

    args = (sums_p, counts_p, atoms_p, st_row, bsum, fold) + tuple(params)
    atoms_new, state_new = pl.pallas_call(
        functools.partial(_node_body, e_edges, n_atoms),
        grid=(),
        in_specs=[fs(a) for a in args],
        out_specs=[pl.BlockSpec((n4, 128), lambda: (0, 0)),
                   pl.BlockSpec((8, 32), lambda: (0, 0))],
        out_shape=[jax.ShapeDtypeStruct((n4, 128), jnp.float32),
                   jax.ShapeDtypeStruct((8, 32), jnp.float32)],
    )(*args)
    return atoms_new, state_new


def kernel(bonds, bond_atom_1, bond_atom_2, atoms, state,
           W_e1, b_e1, W_e2, b_e2, W_e3, b_e3,
           W_v1, b_v1, W_v2, b_v2, W_v3, b_v3,
           W_u1, b_u1, W_u2, b_u2, W_u3, b_u3):
    b, e, d = bonds.shape
    n = atoms.shape[1]
    e4, n4 = e // 4, n // 4
    bonds2 = bonds.reshape(e, d)
    atoms2 = atoms.reshape(n, d)
    idx1 = bond_atom_1.reshape(e).astype(jnp.int32)
    idx2 = bond_atom_2.reshape(e).astype(jnp.int32)
    st_row = state.reshape(1, d)

    a1g, a2g = _sc_gather(atoms2, idx1, idx2)

    bonds_new, bsum = _tc_edge_mlp(
        a1g.reshape(e4, 128), a2g.reshape(e4, 128), bonds2.reshape(e4, 128),
        st_row,
        W_e1[0:32], W_e1[32:64], W_e1[64:96], W_e1[96:128], b_e1,
        W_e2, b_e2, W_e3, b_e3)

    sums, counts = _sc_scatter(n, idx1, bonds_new.reshape(e, 32))

    bf = jnp.bfloat16
    node_params = (
        _bd4(W_v1[0:32]).astype(bf), _bd4(W_v1[32:64]).astype(bf),
        W_v1[64:96], b_v1.reshape(1, -1),
        _bd4(W_v2).astype(bf), jnp.tile(b_v2, 4)[None, :],
        _bd4(W_v3).astype(bf), jnp.tile(b_v3, 4)[None, :],
        W_u1[0:32], W_u1[32:64], W_u1[64:96], b_u1.reshape(1, -1),
        W_u2, b_u2.reshape(1, -1), W_u3, b_u3.reshape(1, -1))
    atoms_new, state_new = _tc_node_state(
        float(e), float(n), sums.reshape(2, n4, 128),
        counts.reshape(2, n4, 128), atoms2.reshape(n4, 128), st_row, bsum,
        node_params)

    return (bonds_new.reshape(b, e, 32),
            atoms_new.reshape(b, n, 32),
            state_new[0:1, :].reshape(b, 1, 32))
